# baseline (speedup 1.0000x reference)
import functools

import jax
import jax.numpy as jnp
from jax import lax
from jax.experimental import pallas as pl
from jax.experimental.pallas import tpu as pltpu
from jax.experimental.pallas import tpu_sc as plsc

N = 10000
E = 160000
D = 256
K = 3

NC, NS, L = 2, 16, 16
NT = NC * NS
N_PAD = 10240
RPT = N_PAD // NT
SCHUNK = 4000
NSC = E // SCHUNK
CAPO = 15360
CHUNK = 48
DBLK = 512

_MESH = plsc.VectorSubcoreMesh(core_axis_name="c", subcore_axis_name="s")
_CP = pltpu.CompilerParams(needs_layout_passes=False)


def _wid():
    return lax.axis_index("s") * NC + lax.axis_index("c")


def _iota16():
    return lax.iota(jnp.int32, 16)


def _mm_body(x_ref, w_ref, o_ref):
    o_ref[...] = lax.dot_general(
        x_ref[...], w_ref[...], (((1,), (1,)), ((), ())),
        preferred_element_type=jnp.float32,
    )


def _tc_matmul(x_pad, W):
    blk = 1024
    return pl.pallas_call(
        _mm_body,
        grid=(N_PAD // blk,),
        in_specs=[
            pl.BlockSpec((blk, D), lambda i: (i, 0)),
            pl.BlockSpec((D, D), lambda i: (0, 0)),
        ],
        out_specs=pl.BlockSpec((blk, D), lambda i: (i, 0)),
        out_shape=jax.ShapeDtypeStruct((N_PAD, D), jnp.float32),
    )(x_pad, W)


def _scale_body(h_ref, s_ref, o_ref):
    o_ref[...] = h_ref[...] * s_ref[...]


def _scale_bias_body(h_ref, s_ref, b_ref, o_ref):
    o_ref[...] = h_ref[...] * s_ref[...] + b_ref[...]


def _tc_scale(h, s_col, b_row=None):
    blk = 1024
    in_specs = [
        pl.BlockSpec((blk, D), lambda i: (i, 0)),
        pl.BlockSpec((blk, 1), lambda i: (i, 0)),
    ]
    body = _scale_body
    args = (h, s_col)
    if b_row is not None:
        in_specs.append(pl.BlockSpec((1, D), lambda i: (0, 0)))
        body = _scale_bias_body
        args = (h, s_col, b_row)
    return pl.pallas_call(
        body,
        grid=(N_PAD // blk,),
        in_specs=in_specs,
        out_specs=pl.BlockSpec((blk, D), lambda i: (i, 0)),
        out_shape=jax.ShapeDtypeStruct((N_PAD, D), jnp.float32),
    )(*args)


@functools.partial(
    pl.kernel,
    out_type=(
        jax.ShapeDtypeStruct((NT * CAPO,), jnp.int32),
        jax.ShapeDtypeStruct((NT * CAPO,), jnp.int32),
        jax.ShapeDtypeStruct((NT * L,), jnp.int32),
        jax.ShapeDtypeStruct((N_PAD,), jnp.float32),
        jax.ShapeDtypeStruct((N_PAD,), jnp.float32),
    ),
    mesh=_MESH,
    compiler_params=_CP,
    scratch_types=[
        pltpu.VMEM((SCHUNK,), jnp.int32),
        pltpu.VMEM((SCHUNK,), jnp.int32),
        pltpu.VMEM((SCHUNK,), jnp.int32),
        pltpu.VMEM((SCHUNK,), jnp.int32),
        pltpu.VMEM((CAPO,), jnp.int32),
        pltpu.VMEM((CAPO,), jnp.int32),
        pltpu.VMEM((RPT,), jnp.float32),
        pltpu.VMEM((RPT,), jnp.float32),
        pltpu.VMEM((RPT,), jnp.float32),
        pltpu.VMEM((L,), jnp.int32),
        pltpu.SemaphoreType.DMA,
        pltpu.SemaphoreType.DMA,
    ],
)
def _sc_prep(src_hbm, dst_hbm, lsrc_hbm, ldst_hbm, lcnt_hbm,
             dinv_hbm, dinv2_hbm,
             ss0, sd0, ss1, sd1, bs, bd, hist, s1v, s2v, cnt_v,
             sem0, sem1):
    t = _wid()
    iota = _iota16()
    lo = t * RPT
    ones = jnp.full((L,), 1.0, jnp.float32)

    def zfill(k, _):
        hist[pl.ds(k * L, L)] = jnp.zeros((L,), jnp.float32)
        return 0
    lax.fori_loop(0, RPT // L, zfill, 0)

    stage = ((ss0, sd0, sem0), (ss1, sd1, sem1))

    def start(p, ch):
        sb, db, sem = stage[p]
        sl = pl.ds(ch * SCHUNK, SCHUNK)
        pltpu.make_async_copy(src_hbm.at[sl], sb, sem).start()
        pltpu.make_async_copy(dst_hbm.at[sl], db, sem).start()

    def wait(p):
        sb, db, sem = stage[p]
        pltpu.make_async_copy(src_hbm.at[pl.ds(0, SCHUNK)], sb, sem).wait()
        pltpu.make_async_copy(dst_hbm.at[pl.ds(0, SCHUNK)], db, sem).wait()

    start(0, 0)
    start(1, 1)

    def outer(ci2, cnt):
        for p in range(2):
            ch = ci2 * 2 + p
            wait(p)
            sb, db, _ = stage[p]

            def scan(k, cnt):
                sv = sb[pl.ds(k * L, L)]
                dv = db[pl.ds(k * L, L)]
                dl = dv - lo
                m = (dl >= 0) & (dl < RPT)
                plsc.addupdate_scatter(hist, [dl], ones, mask=m)
                inc = m.astype(jnp.int32)
                pos = cnt + jnp.cumsum(inc) - 1
                plsc.store_scatter(bs, [pos], sv, mask=m)
                plsc.store_scatter(bd, [pos], dl, mask=m)
                return cnt + jnp.sum(inc)

            cnt = lax.fori_loop(0, SCHUNK // L, scan, cnt)

            @pl.when(ch + 2 < NSC)
            def _():
                start(p, ch + 2)
        return cnt

    cnt = lax.fori_loop(0, NSC // 2, outer, jnp.int32(0))

    padded = ((cnt + CHUNK - 1) // CHUNK) * CHUNK
    for q in range(CHUNK // L):
        pos = cnt + q * L + iota
        plsc.store_scatter(bs, [pos], N + ((t * 8 + q * L + iota) & 127),
                           mask=pos < padded)
        plsc.store_scatter(bd, [pos], iota + q * L, mask=pos < padded)

    cnt_v[...] = jnp.zeros((L,), jnp.int32) + padded
    pltpu.sync_copy(cnt_v, lcnt_hbm.at[pl.ds(t * L, L)])

    nblk = (padded + DBLK - 1) // DBLK

    def flush(bk, _):
        sl = pl.ds(bk * DBLK, DBLK)
        osl = pl.ds(t * CAPO + bk * DBLK, DBLK)
        pltpu.sync_copy(bs.at[sl], lsrc_hbm.at[osl])
        pltpu.sync_copy(bd.at[sl], ldst_hbm.at[osl])
        return 0
    lax.fori_loop(0, nblk, flush, 0)

    def newton(k, _):
        sl = pl.ds(k * L, L)
        d = hist[sl] + 1.0
        i = plsc.bitcast(d, jnp.int32)
        y = plsc.bitcast(jnp.int32(0x5F3759DF) - (i >> 1), jnp.float32)
        for _ in range(4):
            y = y * (1.5 - 0.5 * d * y * y)
        s1v[sl] = y
        s2v[sl] = y * y
        return 0
    lax.fori_loop(0, RPT // L, newton, 0)

    pltpu.sync_copy(s1v, dinv_hbm.at[pl.ds(lo, RPT)])
    pltpu.sync_copy(s2v, dinv2_hbm.at[pl.ds(lo, RPT)])


def _make_prop(last):
  @functools.partial(
    pl.kernel,
    out_type=jax.ShapeDtypeStruct((N_PAD, D), jnp.float32),
    mesh=_MESH,
    compiler_params=_CP,
    scratch_types=[
        pltpu.VMEM((RPT, D), jnp.float32),
        pltpu.VMEM((CAPO,), jnp.int32),
        pltpu.VMEM((CHUNK,), jnp.int32),
        pltpu.VMEM((CHUNK,), jnp.int32),
        pltpu.VMEM((CHUNK, D), jnp.float32),
        pltpu.VMEM((CHUNK, D), jnp.float32),
        pltpu.VMEM((L,), jnp.int32),
        pltpu.VMEM((RPT,), jnp.float32),
        pltpu.VMEM((D,), jnp.float32),
        pltpu.SemaphoreType.DMA,
        pltpu.SemaphoreType.DMA,
    ],
  )
  def _sc_prop(g_hbm, lsrc_hbm, ldst_hbm, lcnt_hbm, scale_hbm, b_hbm, out_hbm,
               acc, sidx, di0, di1, rows0, rows1, cnt_v, sv, bv, sem0, sem1):
    wid = _wid()
    rbase = wid * RPT

    pltpu.sync_copy(lcnt_hbm.at[pl.ds(wid * L, L)], cnt_v)
    nch = cnt_v[...][0] // CHUNK

    pltpu.sync_copy(lsrc_hbm.at[pl.ds(wid * CAPO, CAPO)], sidx)
    pltpu.sync_copy(scale_hbm.at[pl.ds(rbase, RPT)], sv)
    if last:
        pltpu.sync_copy(b_hbm, bv)
    pltpu.sync_copy(g_hbm.at[pl.ds(rbase, RPT)], acc)

    stage = ((di0, rows0, sem0), (di1, rows1, sem1))

    def start(p, ci):
        di, rows, sem = stage[p]
        pltpu.make_async_copy(
            ldst_hbm.at[pl.ds(wid * CAPO + ci * CHUNK, CHUNK)], di, sem
        ).start()
        pltpu.make_async_copy(
            g_hbm.at[sidx.at[pl.ds(ci * CHUNK, CHUNK)]], rows, sem
        ).start()

    def wait(p):
        di, rows, sem = stage[p]
        pltpu.make_async_copy(
            ldst_hbm.at[pl.ds(wid * CAPO, CHUNK)], di, sem).wait()
        pltpu.make_async_copy(
            g_hbm.at[sidx.at[pl.ds(0, CHUNK)]], rows, sem).wait()

    start(0, 0)

    @pl.when(nch > 1)
    def _():
        start(1, 1)

    def run(ci2, _):
        for p in range(2):
            ci = ci2 * 2 + p

            @pl.when(ci < nch)
            def _():
                di, rows, sem = stage[p]
                wait(p)

                def qloop(q, _):
                    dlv = di[pl.ds(q * L, L)]
                    for r in range(L):
                        dl = dlv[r]
                        e = q * L + r
                        vs = [rows[e, pl.ds(j * L, L)] for j in range(D // L)]
                        for j in range(D // L):
                            plsc.addupdate(acc.at[dl, pl.ds(j * L, L)], vs[j])
                    return 0
                lax.fori_loop(0, CHUNK // L, qloop, 0)

                @pl.when(ci + 2 < nch)
                def _():
                    start(p, ci + 2)
        return 0
    lax.fori_loop(0, (nch + 1) // 2, run, 0)

    def grp(q, _):
        s16 = sv[pl.ds(q * L, L)]
        for r in range(L):
            s = s16[r]
            row = q * L + r
            for j in range(D // L):
                sl = pl.ds(j * L, L)
                if last:
                    acc[row, sl] = acc[row, sl] * s + bv[sl]
                else:
                    acc[row, sl] = acc[row, sl] * s
        return 0
    lax.fori_loop(0, RPT // L, grp, 0)

    pltpu.sync_copy(acc, out_hbm.at[pl.ds(rbase, RPT)])

  return _sc_prop


_prop_mid = _make_prop(last=False)
_prop_last = _make_prop(last=True)


def kernel(x, edge_index, W, b):
    ei = edge_index.astype(jnp.int32)
    src_all, dst_all = ei[0], ei[1]
    x_pad = jnp.pad(x, ((0, N_PAD - N), (0, 0)))
    b_row = b.reshape(1, D)

    h0 = _tc_matmul(x_pad, W)
    lsrc, ldst, lcnt, dinv, dinv2 = _sc_prep(src_all, dst_all)

    g = _tc_scale(h0, dinv.reshape(N_PAD, 1))
    for _ in range(K - 1):
        g = _prop_mid(g, lsrc, ldst, lcnt, dinv2, b)
    out = _prop_last(g, lsrc, ldst, lcnt, dinv, b)
    return out[:N]


# speedup vs baseline: 1.1475x; 1.1475x over previous
import functools

import jax
import jax.numpy as jnp
from jax import lax
from jax.experimental import pallas as pl
from jax.experimental.pallas import tpu as pltpu
from jax.experimental.pallas import tpu_sc as plsc

N = 10000
E = 160000
D = 256
K = 3

NC, NS, L = 2, 16, 16
NT = NC * NS
N_PAD = 10240
RPT = N_PAD // NT
ES = E // NT
CAPO = 15360
CHUNK = 48
DBLK = 512

_MESH = plsc.VectorSubcoreMesh(core_axis_name="c", subcore_axis_name="s")
_CP = pltpu.CompilerParams(needs_layout_passes=False)


def _wid():
    return lax.axis_index("s") * NC + lax.axis_index("c")


def _iota16():
    return lax.iota(jnp.int32, 16)


def _mm_body(x_ref, w_ref, o_ref):
    o_ref[...] = lax.dot_general(
        x_ref[...], w_ref[...], (((1,), (1,)), ((), ())),
        preferred_element_type=jnp.float32,
    )


def _tc_matmul(x_pad, W):
    blk = 1024
    return pl.pallas_call(
        _mm_body,
        grid=(N_PAD // blk,),
        in_specs=[
            pl.BlockSpec((blk, D), lambda i: (i, 0)),
            pl.BlockSpec((D, D), lambda i: (0, 0)),
        ],
        out_specs=pl.BlockSpec((blk, D), lambda i: (i, 0)),
        out_shape=jax.ShapeDtypeStruct((N_PAD, D), jnp.float32),
    )(x_pad, W)


def _scale_body(h_ref, s_ref, o_ref):
    o_ref[...] = h_ref[...] * s_ref[...]


def _scale_bias_body(h_ref, s_ref, b_ref, o_ref):
    o_ref[...] = h_ref[...] * s_ref[...] + b_ref[...]


def _tc_scale(h, s_col, b_row=None):
    blk = 1024
    in_specs = [
        pl.BlockSpec((blk, D), lambda i: (i, 0)),
        pl.BlockSpec((blk, 1), lambda i: (i, 0)),
    ]
    body = _scale_body
    args = (h, s_col)
    if b_row is not None:
        in_specs.append(pl.BlockSpec((1, D), lambda i: (0, 0)))
        body = _scale_bias_body
        args = (h, s_col, b_row)
    return pl.pallas_call(
        body,
        grid=(N_PAD // blk,),
        in_specs=in_specs,
        out_specs=pl.BlockSpec((blk, D), lambda i: (i, 0)),
        out_shape=jax.ShapeDtypeStruct((N_PAD, D), jnp.float32),
    )(*args)


NQ = 4
QRNG = N_PAD // NQ
QCAP = 5120


@functools.partial(
    pl.kernel,
    out_type=(
        jax.ShapeDtypeStruct((NQ * NT * QCAP,), jnp.int32),
        jax.ShapeDtypeStruct((NQ * NT * QCAP,), jnp.int32),
        jax.ShapeDtypeStruct((NQ * NT * L,), jnp.int32),
    ),
    mesh=_MESH,
    compiler_params=_CP,
    scratch_types=[
        pltpu.VMEM((ES + 8,), jnp.int32),
        pltpu.VMEM((ES + 8,), jnp.int32),
        pltpu.VMEM((NQ * QCAP,), jnp.int32),
        pltpu.VMEM((NQ * QCAP,), jnp.int32),
        pltpu.VMEM((L,), jnp.int32),
    ],
)
def _sc_prep_a(src_hbm, dst_hbm, qsrc_hbm, qdst_hbm, qcnt_hbm,
               ssrc, sdst, qs, qd, cnt_v):
    t = _wid()
    iota = _iota16()

    pltpu.sync_copy(src_hbm.at[pl.ds(t * ES, ES)], ssrc.at[pl.ds(0, ES)])
    pltpu.sync_copy(dst_hbm.at[pl.ds(t * ES, ES)], sdst.at[pl.ds(0, ES)])
    plsc.store_scatter(ssrc, [iota + ES], jnp.full((L,), N, jnp.int32) + iota,
                       mask=iota < 8)
    plsc.store_scatter(sdst, [iota + ES], jnp.full((L,), N, jnp.int32) + iota,
                       mask=iota < 8)

    def scan(k, carry):
        sv = ssrc[pl.ds(k * L, L)]
        dv = sdst[pl.ds(k * L, L)]
        out = []
        for q in range(NQ):
            m = (dv >= q * QRNG) & (dv < (q + 1) * QRNG)
            inc = m.astype(jnp.int32)
            pos = carry[q] + jnp.cumsum(inc) - 1
            plsc.store_scatter(qs, [q * QCAP + pos], sv, mask=m)
            plsc.store_scatter(qd, [q * QCAP + pos], dv, mask=m)
            out.append(carry[q] + jnp.sum(inc))
        return tuple(out)

    nvec = (ES + 8) // L
    cnts = lax.fori_loop(0, nvec, scan, (jnp.int32(0),) * NQ)

    for q in range(NQ):
        cnt = cnts[q]
        padded = ((cnt + L - 1) // L) * L
        pos = cnt + iota
        plsc.store_scatter(qs, [q * QCAP + pos],
                           jnp.full((L,), N, jnp.int32) + iota,
                           mask=pos < padded)
        plsc.store_scatter(qd, [q * QCAP + pos],
                           jnp.full((L,), N, jnp.int32) + iota,
                           mask=pos < padded)
        cnt_v[...] = jnp.zeros((L,), jnp.int32) + padded
        pltpu.sync_copy(cnt_v, qcnt_hbm.at[pl.ds((q * NT + t) * L, L)])

        nblk = (padded + DBLK - 1) // DBLK

        def flush(bk, _):
            sl = pl.ds(q * QCAP + bk * DBLK, DBLK)
            osl = pl.ds((q * NT + t) * QCAP + bk * DBLK, DBLK)
            pltpu.sync_copy(qs.at[sl], qsrc_hbm.at[osl])
            pltpu.sync_copy(qd.at[sl], qdst_hbm.at[osl])
            return 0
        lax.fori_loop(0, nblk, flush, 0)


@functools.partial(
    pl.kernel,
    out_type=(
        jax.ShapeDtypeStruct((NT * CAPO,), jnp.int32),
        jax.ShapeDtypeStruct((NT * CAPO,), jnp.int32),
        jax.ShapeDtypeStruct((NT * L,), jnp.int32),
        jax.ShapeDtypeStruct((N_PAD,), jnp.float32),
        jax.ShapeDtypeStruct((N_PAD,), jnp.float32),
    ),
    mesh=_MESH,
    compiler_params=_CP,
    scratch_types=[
        pltpu.VMEM((QCAP,), jnp.int32),
        pltpu.VMEM((QCAP,), jnp.int32),
        pltpu.VMEM((QCAP,), jnp.int32),
        pltpu.VMEM((QCAP,), jnp.int32),
        pltpu.VMEM((NT * L,), jnp.int32),
        pltpu.VMEM((CAPO,), jnp.int32),
        pltpu.VMEM((CAPO,), jnp.int32),
        pltpu.VMEM((RPT,), jnp.float32),
        pltpu.VMEM((RPT,), jnp.float32),
        pltpu.VMEM((RPT,), jnp.float32),
        pltpu.VMEM((L,), jnp.int32),
        pltpu.SemaphoreType.DMA,
        pltpu.SemaphoreType.DMA,
    ],
)
def _sc_prep(qsrc_hbm, qdst_hbm, qcnt_hbm, lsrc_hbm, ldst_hbm, lcnt_hbm,
             dinv_hbm, dinv2_hbm,
             ss0, sd0, ss1, sd1, cbuf, bs, bd, hist, s1v, s2v, cnt_v,
             sem0, sem1):
    t = _wid()
    iota = _iota16()
    lo = t * RPT
    q = t // (NT // NQ)
    ones = jnp.full((L,), 1.0, jnp.float32)

    pltpu.sync_copy(qcnt_hbm.at[pl.ds(q * NT * L, NT * L)], cbuf)

    def zfill(k, _):
        hist[pl.ds(k * L, L)] = jnp.zeros((L,), jnp.float32)
        return 0
    lax.fori_loop(0, RPT // L, zfill, 0)

    stage = ((ss0, sd0, sem0), (ss1, sd1, sem1))

    def start(p, s):
        sb, db, sem = stage[p]
        sl = pl.ds((q * NT + s) * QCAP, QCAP)
        pltpu.make_async_copy(qsrc_hbm.at[sl], sb, sem).start()
        pltpu.make_async_copy(qdst_hbm.at[sl], db, sem).start()

    def wait(p):
        sb, db, sem = stage[p]
        pltpu.make_async_copy(qsrc_hbm.at[pl.ds(0, QCAP)], sb, sem).wait()
        pltpu.make_async_copy(qdst_hbm.at[pl.ds(0, QCAP)], db, sem).wait()

    start(0, 0)
    start(1, 1)

    cnt = jnp.int32(0)
    for s in range(NT):
        p = s % 2
        wait(p)
        sb, db, _ = stage[p]
        nv = cbuf[pl.ds(s * L, L)][0] // L

        def scan(k, cnt):
            sv = sb[pl.ds(k * L, L)]
            dv = db[pl.ds(k * L, L)]
            dl = dv - lo
            m = (dl >= 0) & (dl < RPT)
            plsc.addupdate_scatter(hist, [dl], ones, mask=m)
            inc = m.astype(jnp.int32)
            pos = cnt + jnp.cumsum(inc) - 1
            plsc.store_scatter(bs, [pos], sv, mask=m)
            plsc.store_scatter(bd, [pos], dl, mask=m)
            return cnt + jnp.sum(inc)

        cnt = lax.fori_loop(0, nv, scan, cnt)
        if s + 2 < NT:
            start(p, s + 2)

    padded = ((cnt + CHUNK - 1) // CHUNK) * CHUNK
    for q in range(CHUNK // L):
        pos = cnt + q * L + iota
        plsc.store_scatter(bs, [pos], N + ((t * 8 + q * L + iota) & 127),
                           mask=pos < padded)
        plsc.store_scatter(bd, [pos], iota + q * L, mask=pos < padded)

    cnt_v[...] = jnp.zeros((L,), jnp.int32) + padded
    pltpu.sync_copy(cnt_v, lcnt_hbm.at[pl.ds(t * L, L)])

    nblk = (padded + DBLK - 1) // DBLK

    def flush(bk, _):
        sl = pl.ds(bk * DBLK, DBLK)
        osl = pl.ds(t * CAPO + bk * DBLK, DBLK)
        pltpu.sync_copy(bs.at[sl], lsrc_hbm.at[osl])
        pltpu.sync_copy(bd.at[sl], ldst_hbm.at[osl])
        return 0
    lax.fori_loop(0, nblk, flush, 0)

    def newton(k, _):
        sl = pl.ds(k * L, L)
        d = hist[sl] + 1.0
        i = plsc.bitcast(d, jnp.int32)
        y = plsc.bitcast(jnp.int32(0x5F3759DF) - (i >> 1), jnp.float32)
        for _ in range(4):
            y = y * (1.5 - 0.5 * d * y * y)
        s1v[sl] = y
        s2v[sl] = y * y
        return 0
    lax.fori_loop(0, RPT // L, newton, 0)

    pltpu.sync_copy(s1v, dinv_hbm.at[pl.ds(lo, RPT)])
    pltpu.sync_copy(s2v, dinv2_hbm.at[pl.ds(lo, RPT)])


def _make_prop(last):
  @functools.partial(
    pl.kernel,
    out_type=jax.ShapeDtypeStruct((N_PAD, D), jnp.float32),
    mesh=_MESH,
    compiler_params=_CP,
    scratch_types=[
        pltpu.VMEM((RPT, D), jnp.float32),
        pltpu.VMEM((CAPO,), jnp.int32),
        pltpu.VMEM((CHUNK,), jnp.int32),
        pltpu.VMEM((CHUNK,), jnp.int32),
        pltpu.VMEM((CHUNK, D), jnp.float32),
        pltpu.VMEM((CHUNK, D), jnp.float32),
        pltpu.VMEM((L,), jnp.int32),
        pltpu.VMEM((RPT,), jnp.float32),
        pltpu.VMEM((D,), jnp.float32),
        pltpu.SemaphoreType.DMA,
        pltpu.SemaphoreType.DMA,
    ],
  )
  def _sc_prop(g_hbm, lsrc_hbm, ldst_hbm, lcnt_hbm, scale_hbm, b_hbm, out_hbm,
               acc, sidx, di0, di1, rows0, rows1, cnt_v, sv, bv, sem0, sem1):
    wid = _wid()
    rbase = wid * RPT

    pltpu.sync_copy(lcnt_hbm.at[pl.ds(wid * L, L)], cnt_v)
    nch = cnt_v[...][0] // CHUNK

    pltpu.sync_copy(lsrc_hbm.at[pl.ds(wid * CAPO, CAPO)], sidx)
    pltpu.sync_copy(scale_hbm.at[pl.ds(rbase, RPT)], sv)
    if last:
        pltpu.sync_copy(b_hbm, bv)
    pltpu.sync_copy(g_hbm.at[pl.ds(rbase, RPT)], acc)

    stage = ((di0, rows0, sem0), (di1, rows1, sem1))

    def start(p, ci):
        di, rows, sem = stage[p]
        pltpu.make_async_copy(
            ldst_hbm.at[pl.ds(wid * CAPO + ci * CHUNK, CHUNK)], di, sem
        ).start()
        pltpu.make_async_copy(
            g_hbm.at[sidx.at[pl.ds(ci * CHUNK, CHUNK)]], rows, sem
        ).start()

    def wait(p):
        di, rows, sem = stage[p]
        pltpu.make_async_copy(
            ldst_hbm.at[pl.ds(wid * CAPO, CHUNK)], di, sem).wait()
        pltpu.make_async_copy(
            g_hbm.at[sidx.at[pl.ds(0, CHUNK)]], rows, sem).wait()

    start(0, 0)

    @pl.when(nch > 1)
    def _():
        start(1, 1)

    def run(ci2, _):
        for p in range(2):
            ci = ci2 * 2 + p

            @pl.when(ci < nch)
            def _():
                di, rows, sem = stage[p]
                wait(p)

                def qloop(q, _):
                    dlv = di[pl.ds(q * L, L)]
                    for r in range(L):
                        dl = dlv[r]
                        e = q * L + r
                        vs = [rows[e, pl.ds(j * L, L)] for j in range(D // L)]
                        for j in range(D // L):
                            plsc.addupdate(acc.at[dl, pl.ds(j * L, L)], vs[j])
                    return 0
                lax.fori_loop(0, CHUNK // L, qloop, 0)

                @pl.when(ci + 2 < nch)
                def _():
                    start(p, ci + 2)
        return 0
    lax.fori_loop(0, (nch + 1) // 2, run, 0)

    def grp(q, _):
        s16 = sv[pl.ds(q * L, L)]
        for r in range(L):
            s = s16[r]
            row = q * L + r
            for j in range(D // L):
                sl = pl.ds(j * L, L)
                if last:
                    acc[row, sl] = acc[row, sl] * s + bv[sl]
                else:
                    acc[row, sl] = acc[row, sl] * s
        return 0
    lax.fori_loop(0, RPT // L, grp, 0)

    pltpu.sync_copy(acc, out_hbm.at[pl.ds(rbase, RPT)])

  return _sc_prop


_prop_mid = _make_prop(last=False)
_prop_last = _make_prop(last=True)


def kernel(x, edge_index, W, b):
    ei = edge_index.astype(jnp.int32)
    src_all, dst_all = ei[0], ei[1]
    x_pad = jnp.pad(x, ((0, N_PAD - N), (0, 0)))
    b_row = b.reshape(1, D)

    h0 = _tc_matmul(x_pad, W)
    qsrc, qdst, qcnt = _sc_prep_a(src_all, dst_all)
    lsrc, ldst, lcnt, dinv, dinv2 = _sc_prep(qsrc, qdst, qcnt)

    g = _tc_scale(h0, dinv.reshape(N_PAD, 1))
    for _ in range(K - 1):
        g = _prop_mid(g, lsrc, ldst, lcnt, dinv2, b)
    out = _prop_last(g, lsrc, ldst, lcnt, dinv, b)
    return out[:N]


# speedup vs baseline: 1.2595x; 1.0976x over previous
import functools

import jax
import jax.numpy as jnp
from jax import lax
from jax.experimental import pallas as pl
from jax.experimental.pallas import tpu as pltpu
from jax.experimental.pallas import tpu_sc as plsc

N = 10000
E = 160000
D = 256
K = 3

NC, NS, L = 2, 16, 16
NT = NC * NS
N_PAD = 10240
RPT = N_PAD // NT
ES = E // NT
CAPO = 14336
CHUNK = 64
DBLK = 512

_MESH = plsc.VectorSubcoreMesh(core_axis_name="c", subcore_axis_name="s")
_CP = pltpu.CompilerParams(needs_layout_passes=False)


def _wid():
    return lax.axis_index("s") * NC + lax.axis_index("c")


def _iota16():
    return lax.iota(jnp.int32, 16)


def _mm_body(x_ref, w_ref, o_ref):
    o_ref[...] = lax.dot_general(
        x_ref[...], w_ref[...], (((1,), (1,)), ((), ())),
        preferred_element_type=jnp.float32,
    )


def _tc_matmul(x_pad, W):
    blk = 1024
    return pl.pallas_call(
        _mm_body,
        grid=(N_PAD // blk,),
        in_specs=[
            pl.BlockSpec((blk, D), lambda i: (i, 0)),
            pl.BlockSpec((D, D), lambda i: (0, 0)),
        ],
        out_specs=pl.BlockSpec((blk, D), lambda i: (i, 0)),
        out_shape=jax.ShapeDtypeStruct((N_PAD, D), jnp.float32),
    )(x_pad, W)


def _scale_body(h_ref, s_ref, o_ref):
    o_ref[...] = h_ref[...] * s_ref[...]


def _scale_bias_body(h_ref, s_ref, b_ref, o_ref):
    o_ref[...] = h_ref[...] * s_ref[...] + b_ref[...]


def _tc_scale(h, s_col, b_row=None):
    blk = 1024
    in_specs = [
        pl.BlockSpec((blk, D), lambda i: (i, 0)),
        pl.BlockSpec((blk, 1), lambda i: (i, 0)),
    ]
    body = _scale_body
    args = (h, s_col)
    if b_row is not None:
        in_specs.append(pl.BlockSpec((1, D), lambda i: (0, 0)))
        body = _scale_bias_body
        args = (h, s_col, b_row)
    return pl.pallas_call(
        body,
        grid=(N_PAD // blk,),
        in_specs=in_specs,
        out_specs=pl.BlockSpec((blk, D), lambda i: (i, 0)),
        out_shape=jax.ShapeDtypeStruct((N_PAD, D), jnp.float32),
    )(*args)


NQ = 4
QRNG = N_PAD // NQ
QCAP = 5120


@functools.partial(
    pl.kernel,
    out_type=(
        jax.ShapeDtypeStruct((NQ * NT * QCAP,), jnp.int32),
        jax.ShapeDtypeStruct((NQ * NT * QCAP,), jnp.int32),
        jax.ShapeDtypeStruct((NQ * NT * L,), jnp.int32),
    ),
    mesh=_MESH,
    compiler_params=_CP,
    scratch_types=[
        pltpu.VMEM((ES + 8,), jnp.int32),
        pltpu.VMEM((ES + 8,), jnp.int32),
        pltpu.VMEM((NQ * QCAP,), jnp.int32),
        pltpu.VMEM((NQ * QCAP,), jnp.int32),
        pltpu.VMEM((L,), jnp.int32),
    ],
)
def _sc_prep_a(src_hbm, dst_hbm, qsrc_hbm, qdst_hbm, qcnt_hbm,
               ssrc, sdst, qs, qd, cnt_v):
    t = _wid()
    iota = _iota16()

    pltpu.sync_copy(src_hbm.at[pl.ds(t * ES, ES)], ssrc.at[pl.ds(0, ES)])
    pltpu.sync_copy(dst_hbm.at[pl.ds(t * ES, ES)], sdst.at[pl.ds(0, ES)])
    plsc.store_scatter(ssrc, [iota + ES], jnp.full((L,), N, jnp.int32) + iota,
                       mask=iota < 8)
    plsc.store_scatter(sdst, [iota + ES], jnp.full((L,), N, jnp.int32) + iota,
                       mask=iota < 8)

    def scan(k, carry):
        sv = ssrc[pl.ds(k * L, L)]
        dv = sdst[pl.ds(k * L, L)]
        out = []
        for q in range(NQ):
            m = (dv >= q * QRNG) & (dv < (q + 1) * QRNG)
            inc = m.astype(jnp.int32)
            pos = carry[q] + jnp.cumsum(inc) - 1
            plsc.store_scatter(qs, [q * QCAP + pos], sv, mask=m)
            plsc.store_scatter(qd, [q * QCAP + pos], dv, mask=m)
            out.append(carry[q] + jnp.sum(inc))
        return tuple(out)

    nvec = (ES + 8) // L
    cnts = lax.fori_loop(0, nvec, scan, (jnp.int32(0),) * NQ)

    for q in range(NQ):
        cnt = cnts[q]
        padded = ((cnt + L - 1) // L) * L
        pos = cnt + iota
        plsc.store_scatter(qs, [q * QCAP + pos],
                           jnp.full((L,), N, jnp.int32) + iota,
                           mask=pos < padded)
        plsc.store_scatter(qd, [q * QCAP + pos],
                           jnp.full((L,), N, jnp.int32) + iota,
                           mask=pos < padded)
        cnt_v[...] = jnp.zeros((L,), jnp.int32) + padded
        pltpu.sync_copy(cnt_v, qcnt_hbm.at[pl.ds((q * NT + t) * L, L)])

        nblk = (padded + DBLK - 1) // DBLK

        def flush(bk, _):
            sl = pl.ds(q * QCAP + bk * DBLK, DBLK)
            osl = pl.ds((q * NT + t) * QCAP + bk * DBLK, DBLK)
            pltpu.sync_copy(qs.at[sl], qsrc_hbm.at[osl])
            pltpu.sync_copy(qd.at[sl], qdst_hbm.at[osl])
            return 0
        lax.fori_loop(0, nblk, flush, 0)


@functools.partial(
    pl.kernel,
    out_type=(
        jax.ShapeDtypeStruct((NT * CAPO,), jnp.int32),
        jax.ShapeDtypeStruct((NT * CAPO,), jnp.int32),
        jax.ShapeDtypeStruct((NT * L,), jnp.int32),
        jax.ShapeDtypeStruct((N_PAD,), jnp.float32),
        jax.ShapeDtypeStruct((N_PAD,), jnp.float32),
    ),
    mesh=_MESH,
    compiler_params=_CP,
    scratch_types=[
        pltpu.VMEM((QCAP,), jnp.int32),
        pltpu.VMEM((QCAP,), jnp.int32),
        pltpu.VMEM((QCAP,), jnp.int32),
        pltpu.VMEM((QCAP,), jnp.int32),
        pltpu.VMEM((NT * L,), jnp.int32),
        pltpu.VMEM((CAPO,), jnp.int32),
        pltpu.VMEM((CAPO,), jnp.int32),
        pltpu.VMEM((RPT,), jnp.float32),
        pltpu.VMEM((RPT,), jnp.float32),
        pltpu.VMEM((RPT,), jnp.float32),
        pltpu.VMEM((L,), jnp.int32),
        pltpu.SemaphoreType.DMA,
        pltpu.SemaphoreType.DMA,
    ],
)
def _sc_prep(qsrc_hbm, qdst_hbm, qcnt_hbm, lsrc_hbm, ldst_hbm, lcnt_hbm,
             dinv_hbm, dinv2_hbm,
             ss0, sd0, ss1, sd1, cbuf, bs, bd, hist, s1v, s2v, cnt_v,
             sem0, sem1):
    t = _wid()
    iota = _iota16()
    lo = t * RPT
    q = t // (NT // NQ)
    ones = jnp.full((L,), 1.0, jnp.float32)

    pltpu.sync_copy(qcnt_hbm.at[pl.ds(q * NT * L, NT * L)], cbuf)

    def zfill(k, _):
        hist[pl.ds(k * L, L)] = jnp.zeros((L,), jnp.float32)
        return 0
    lax.fori_loop(0, RPT // L, zfill, 0)

    stage = ((ss0, sd0, sem0), (ss1, sd1, sem1))

    def start(p, s):
        sb, db, sem = stage[p]
        sl = pl.ds((q * NT + s) * QCAP, QCAP)
        pltpu.make_async_copy(qsrc_hbm.at[sl], sb, sem).start()
        pltpu.make_async_copy(qdst_hbm.at[sl], db, sem).start()

    def wait(p):
        sb, db, sem = stage[p]
        pltpu.make_async_copy(qsrc_hbm.at[pl.ds(0, QCAP)], sb, sem).wait()
        pltpu.make_async_copy(qdst_hbm.at[pl.ds(0, QCAP)], db, sem).wait()

    start(0, 0)
    start(1, 1)

    cnt = jnp.int32(0)
    for s in range(NT):
        p = s % 2
        wait(p)
        sb, db, _ = stage[p]
        nv = cbuf[pl.ds(s * L, L)][0] // L

        def scan(k, cnt):
            sv = sb[pl.ds(k * L, L)]
            dv = db[pl.ds(k * L, L)]
            dl = dv - lo
            m = (dl >= 0) & (dl < RPT)
            plsc.addupdate_scatter(hist, [dl], ones, mask=m)
            inc = m.astype(jnp.int32)
            pos = cnt + jnp.cumsum(inc) - 1
            plsc.store_scatter(bs, [pos], sv, mask=m)
            plsc.store_scatter(bd, [pos], dl, mask=m)
            return cnt + jnp.sum(inc)

        cnt = lax.fori_loop(0, nv, scan, cnt)
        if s + 2 < NT:
            start(p, s + 2)

    padded = ((cnt + CHUNK - 1) // CHUNK) * CHUNK
    for q in range(CHUNK // L):
        pos = cnt + q * L + iota
        plsc.store_scatter(bs, [pos], N + ((t * 8 + q * L + iota) & 127),
                           mask=pos < padded)
        plsc.store_scatter(bd, [pos], iota + q * L, mask=pos < padded)

    cnt_v[...] = jnp.zeros((L,), jnp.int32) + padded
    pltpu.sync_copy(cnt_v, lcnt_hbm.at[pl.ds(t * L, L)])

    nblk = (padded + DBLK - 1) // DBLK

    def flush(bk, _):
        sl = pl.ds(bk * DBLK, DBLK)
        osl = pl.ds(t * CAPO + bk * DBLK, DBLK)
        pltpu.sync_copy(bs.at[sl], lsrc_hbm.at[osl])
        pltpu.sync_copy(bd.at[sl], ldst_hbm.at[osl])
        return 0
    lax.fori_loop(0, nblk, flush, 0)

    def newton(k, _):
        sl = pl.ds(k * L, L)
        d = hist[sl] + 1.0
        i = plsc.bitcast(d, jnp.int32)
        y = plsc.bitcast(jnp.int32(0x5F3759DF) - (i >> 1), jnp.float32)
        for _ in range(4):
            y = y * (1.5 - 0.5 * d * y * y)
        s1v[sl] = y
        s2v[sl] = y * y
        return 0
    lax.fori_loop(0, RPT // L, newton, 0)

    pltpu.sync_copy(s1v, dinv_hbm.at[pl.ds(lo, RPT)])
    pltpu.sync_copy(s2v, dinv2_hbm.at[pl.ds(lo, RPT)])


def _make_prop(last):
  @functools.partial(
    pl.kernel,
    out_type=jax.ShapeDtypeStruct((N_PAD, D), jnp.float32),
    mesh=_MESH,
    compiler_params=_CP,
    scratch_types=[
        pltpu.VMEM((RPT, D), jnp.float32),
        pltpu.VMEM((CAPO,), jnp.int32),
        pltpu.VMEM((CHUNK,), jnp.int32),
        pltpu.VMEM((CHUNK,), jnp.int32),
        pltpu.VMEM((CHUNK, D), jnp.float32),
        pltpu.VMEM((CHUNK, D), jnp.float32),
        pltpu.VMEM((L,), jnp.int32),
        pltpu.VMEM((RPT,), jnp.float32),
        pltpu.VMEM((D,), jnp.float32),
        pltpu.SemaphoreType.DMA,
        pltpu.SemaphoreType.DMA,
    ],
  )
  def _sc_prop(g_hbm, lsrc_hbm, ldst_hbm, lcnt_hbm, scale_hbm, b_hbm, out_hbm,
               acc, sidx, di0, di1, rows0, rows1, cnt_v, sv, bv, sem0, sem1):
    wid = _wid()
    rbase = wid * RPT

    pltpu.sync_copy(lcnt_hbm.at[pl.ds(wid * L, L)], cnt_v)
    nch = cnt_v[...][0] // CHUNK

    pltpu.sync_copy(lsrc_hbm.at[pl.ds(wid * CAPO, CAPO)], sidx)
    pltpu.sync_copy(scale_hbm.at[pl.ds(rbase, RPT)], sv)
    if last:
        pltpu.sync_copy(b_hbm, bv)
    pltpu.sync_copy(g_hbm.at[pl.ds(rbase, RPT)], acc)

    stage = ((di0, rows0, sem0), (di1, rows1, sem1))

    def start(p, ci):
        di, rows, sem = stage[p]
        pltpu.make_async_copy(
            ldst_hbm.at[pl.ds(wid * CAPO + ci * CHUNK, CHUNK)], di, sem
        ).start()
        pltpu.make_async_copy(
            g_hbm.at[sidx.at[pl.ds(ci * CHUNK, CHUNK)]], rows, sem
        ).start()

    def wait(p):
        di, rows, sem = stage[p]
        pltpu.make_async_copy(
            ldst_hbm.at[pl.ds(wid * CAPO, CHUNK)], di, sem).wait()
        pltpu.make_async_copy(
            g_hbm.at[sidx.at[pl.ds(0, CHUNK)]], rows, sem).wait()

    start(0, 0)

    @pl.when(nch > 1)
    def _():
        start(1, 1)

    def run(ci2, _):
        for p in range(2):
            ci = ci2 * 2 + p

            @pl.when(ci < nch)
            def _():
                di, rows, sem = stage[p]
                wait(p)

                def qloop(q, _):
                    dlv = di[pl.ds(q * L, L)]
                    for r in range(L):
                        dl = dlv[r]
                        e = q * L + r
                        vs = [rows[e, pl.ds(j * L, L)] for j in range(D // L)]
                        for j in range(D // L):
                            plsc.addupdate(acc.at[dl, pl.ds(j * L, L)], vs[j])
                    return 0
                lax.fori_loop(0, CHUNK // L, qloop, 0)

                @pl.when(ci + 2 < nch)
                def _():
                    start(p, ci + 2)
        return 0
    lax.fori_loop(0, (nch + 1) // 2, run, 0)

    bvs = [bv[pl.ds(j * L, L)] for j in range(D // L)] if last else None

    def grp(q, _):
        s16 = sv[pl.ds(q * L, L)]
        for r in range(L):
            s = s16[r]
            row = q * L + r
            for j in range(D // L):
                sl = pl.ds(j * L, L)
                if last:
                    acc[row, sl] = acc[row, sl] * s + bvs[j]
                else:
                    acc[row, sl] = acc[row, sl] * s
        return 0
    lax.fori_loop(0, RPT // L, grp, 0)

    pltpu.sync_copy(acc, out_hbm.at[pl.ds(rbase, RPT)])

  return _sc_prop


_prop_mid = _make_prop(last=False)
_prop_last = _make_prop(last=True)


def kernel(x, edge_index, W, b):
    ei = edge_index.astype(jnp.int32)
    src_all, dst_all = ei[0], ei[1]
    x_pad = jnp.pad(x, ((0, N_PAD - N), (0, 0)))
    b_row = b.reshape(1, D)

    h0 = _tc_matmul(x_pad, W)
    qsrc, qdst, qcnt = _sc_prep_a(src_all, dst_all)
    lsrc, ldst, lcnt, dinv, dinv2 = _sc_prep(qsrc, qdst, qcnt)

    g = _tc_scale(h0, dinv.reshape(N_PAD, 1))
    for _ in range(K - 1):
        g = _prop_mid(g, lsrc, ldst, lcnt, dinv2, b)
    out = _prop_last(g, lsrc, ldst, lcnt, dinv, b)
    return out[:N]
